# Initial kernel scaffold; baseline (speedup 1.0000x reference)
import jax
import jax.numpy as jnp
from jax.experimental import pallas as pl


def kernel(x, mask_embedding):
    raise NotImplementedError("write your pallas kernel here")


# speedup vs baseline: 1.8186x; 1.8186x over previous
import jax
import jax.numpy as jnp
import numpy as np
from jax.experimental import pallas as pl
from jax.experimental.pallas import tpu as pltpu

_MASK_PROB = 0.065
_MASK_LENGTH = 10
_TBLK = 512

_starts_cache = {}


def _span_starts(B, T):
    if (B, T) not in _starts_cache:
        n = int(_MASK_PROB * T / _MASK_LENGTH)
        with jax.ensure_compile_time_eval():
            key = jax.random.key(42)
            rows = []
            for b in range(B):
                kb = jax.random.fold_in(key, b)
                rows.append(np.asarray(jax.random.randint(kb, (n,), 0, T - _MASK_LENGTH)))
        _starts_cache[(B, T)] = np.stack(rows).astype(np.int32)
    return _starts_cache[(B, T)]


def _masked_fill_kernel(starts_ref, x_ref, emb_ref, out_ref, mask_ref, *, tblk, nspans):
    b = pl.program_id(0)
    t0 = pl.program_id(1) * tblk
    trow = jax.lax.broadcasted_iota(jnp.int32, (tblk, 1), 0) + t0
    tlane = jax.lax.broadcasted_iota(jnp.int32, (1, tblk), 1) + t0
    mrow = jnp.zeros((tblk, 1), jnp.bool_)
    mlane = jnp.zeros((1, tblk), jnp.bool_)
    for s in range(nspans):
        st = starts_ref[b, s]
        mrow = mrow | ((trow >= st) & (trow < st + _MASK_LENGTH))
        mlane = mlane | ((tlane >= st) & (tlane < st + _MASK_LENGTH))
    out_ref[0] = jnp.where(mrow, emb_ref[...], x_ref[0])
    mask_ref[0] = mlane.astype(jnp.int32)


def kernel(x, mask_embedding):
    B, T, C = x.shape
    starts = _span_starts(B, T)
    nspans = starts.shape[1]
    tblk = _TBLK

    grid_spec = pltpu.PrefetchScalarGridSpec(
        num_scalar_prefetch=1,
        grid=(B, T // tblk),
        in_specs=[
            pl.BlockSpec((1, tblk, C), lambda b, t, s: (b, t, 0)),
            pl.BlockSpec((1, C), lambda b, t, s: (0, 0)),
        ],
        out_specs=[
            pl.BlockSpec((1, tblk, C), lambda b, t, s: (b, t, 0)),
            pl.BlockSpec((1, 1, tblk), lambda b, t, s: (b, 0, t)),
        ],
    )
    import functools
    body = functools.partial(_masked_fill_kernel, tblk=tblk, nspans=nspans)
    x_masked, mask_i32 = pl.pallas_call(
        body,
        grid_spec=grid_spec,
        out_shape=[
            jax.ShapeDtypeStruct((B, T, C), x.dtype),
            jax.ShapeDtypeStruct((B, 1, T), jnp.int32),
        ],
    )(starts, x, mask_embedding.reshape(1, C))
    return (x_masked, mask_i32.reshape(B, T).astype(bool))


# speedup vs baseline: 1.9402x; 1.0669x over previous
import jax
import jax.numpy as jnp
import numpy as np
from jax.experimental import pallas as pl
from jax.experimental.pallas import tpu as pltpu

_MASK_PROB = 0.065
_MASK_LENGTH = 10
_TBLK = 1024

_starts_cache = {}


def _span_starts(B, T):
    if (B, T) not in _starts_cache:
        n = int(_MASK_PROB * T / _MASK_LENGTH)
        with jax.ensure_compile_time_eval():
            key = jax.random.key(42)
            rows = []
            for b in range(B):
                kb = jax.random.fold_in(key, b)
                rows.append(np.asarray(jax.random.randint(kb, (n,), 0, T - _MASK_LENGTH)))
        _starts_cache[(B, T)] = np.stack(rows).astype(np.int32)
    return _starts_cache[(B, T)]


def _masked_fill_kernel(starts_ref, x_ref, emb_ref, out_ref, mask_ref, *, tblk, nspans):
    b = pl.program_id(0)
    t0 = pl.program_id(1) * tblk
    trow = jax.lax.broadcasted_iota(jnp.int32, (tblk, 1), 0) + t0
    tlane = jax.lax.broadcasted_iota(jnp.int32, (1, tblk), 1) + t0
    mrow = jnp.zeros((tblk, 1), jnp.bool_)
    mlane = jnp.zeros((1, tblk), jnp.bool_)
    for s in range(nspans):
        st = starts_ref[b, s]
        mrow = mrow | ((trow >= st) & (trow < st + _MASK_LENGTH))
        mlane = mlane | ((tlane >= st) & (tlane < st + _MASK_LENGTH))
    out_ref[0] = jnp.where(mrow, emb_ref[...], x_ref[0])
    mask_ref[0] = mlane.astype(jnp.int32)


def kernel(x, mask_embedding):
    B, T, C = x.shape
    starts = _span_starts(B, T)
    nspans = starts.shape[1]
    tblk = _TBLK

    grid_spec = pltpu.PrefetchScalarGridSpec(
        num_scalar_prefetch=1,
        grid=(B, T // tblk),
        in_specs=[
            pl.BlockSpec((1, tblk, C), lambda b, t, s: (b, t, 0)),
            pl.BlockSpec((1, C), lambda b, t, s: (0, 0)),
        ],
        out_specs=[
            pl.BlockSpec((1, tblk, C), lambda b, t, s: (b, t, 0)),
            pl.BlockSpec((1, 1, tblk), lambda b, t, s: (b, 0, t)),
        ],
    )
    import functools
    body = functools.partial(_masked_fill_kernel, tblk=tblk, nspans=nspans)
    x_masked, mask_i32 = pl.pallas_call(
        body,
        grid_spec=grid_spec,
        out_shape=[
            jax.ShapeDtypeStruct((B, T, C), x.dtype),
            jax.ShapeDtypeStruct((B, 1, T), jnp.int32),
        ],
    )(starts, x, mask_embedding.reshape(1, C))
    return (x_masked, mask_i32.reshape(B, T).astype(bool))
